# baseline (speedup 1.0000x reference)
import functools
import jax
import jax.numpy as jnp
from jax import lax
from jax.experimental import pallas as pl
from jax.experimental.pallas import tpu as pltpu
from jax.experimental.pallas import tpu_sc as plsc

P = 0.15

_NC = 2
_NS = 16
_NW = _NC * _NS
_CHUNK = 200
_T = 500
_KMAX = 16
_TAIL = _T - _NW * (_KMAX - 1)
_NBUF = 3
_L = 16


def _sc_mask(x, mask_u):
    n, d = x.shape
    mesh = plsc.VectorSubcoreMesh(core_axis_name="c", subcore_axis_name="s")

    @functools.partial(
        pl.kernel,
        out_type=jax.ShapeDtypeStruct((n, d), x.dtype),
        mesh=mesh,
        scratch_types=[
            pltpu.VMEM((_CHUNK, d), jnp.float32),
            pltpu.VMEM((_CHUNK, d), jnp.float32),
            pltpu.VMEM((_CHUNK, d), jnp.float32),
            pltpu.VMEM((d,), jnp.float32),
            pltpu.SemaphoreType.DMA,
            pltpu.SemaphoreType.DMA,
            pltpu.SemaphoreType.DMA,
            pltpu.SemaphoreType.DMA,
            pltpu.SemaphoreType.DMA,
            pltpu.SemaphoreType.DMA,
        ],
    )
    def run(x_hbm, mask_hbm, out_hbm, b0, b1, b2, mask_v,
            si0, si1, si2, so0, so1, so2):
        wid = lax.axis_index("s") * _NC + lax.axis_index("c")
        bufs = (b0, b1, b2)
        isems = (si0, si1, si2)
        osems = (so0, so1, so2)

        pltpu.sync_copy(mask_hbm, mask_v)
        keep = [
            jnp.where(mask_v[pl.ds(_L * g, _L)] < P, 0.0, 1.0)
            for g in range(d // _L)
        ]

        def rows(k):
            return pl.ds((wid + _NW * k) * _CHUNK, _CHUNK)

        def start_in(k):
            pltpu.async_copy(x_hbm.at[rows(k)], bufs[k % _NBUF],
                             isems[k % _NBUF])

        def wait_in(k):
            pltpu.make_async_copy(x_hbm.at[rows(k)], bufs[k % _NBUF],
                                  isems[k % _NBUF]).wait()

        def start_out(k):
            pltpu.async_copy(bufs[k % _NBUF], out_hbm.at[rows(k)],
                             osems[k % _NBUF])

        def wait_out(k):
            pltpu.make_async_copy(bufs[k % _NBUF], out_hbm.at[rows(k)],
                                  osems[k % _NBUF]).wait()

        def guarded(k, fn):
            if k < _KMAX - 1:
                fn()
            else:
                pl.when(wid < _TAIL)(fn)

        def compute(k):
            buf = bufs[k % _NBUF]

            def row_body(r, carry):
                for g in range(d // _L):
                    sl = pl.ds(_L * g, _L)
                    buf[r, sl] = buf[r, sl] * keep[g]
                return carry

            lax.fori_loop(0, _CHUNK, row_body, 0)

        guarded(0, lambda: start_in(0))
        guarded(1, lambda: start_in(1))
        for k in range(_KMAX):
            def stage(k=k):
                wait_in(k)
                compute(k)
                start_out(k)
            guarded(k, stage)
            if k + 2 < _KMAX:
                if k >= 1:
                    guarded(k - 1, lambda k=k: wait_out(k - 1))
                guarded(k + 2, lambda k=k: start_in(k + 2))
        for k in range(_KMAX - 3, _KMAX):
            guarded(k, lambda k=k: wait_out(k))

    return run(x, mask_u)


def kernel(x, y, mask_u):
    return (_sc_mask(x, mask_u), y)
